# baseline (speedup 1.0000x reference)
import functools

import jax
import jax.numpy as jnp
from jax import lax
from jax.experimental import pallas as pl
from jax.experimental.pallas import tpu as pltpu
from jax.experimental.pallas import tpu_sc as plsc

E = 160000
T = 320000
H = 128
NB = 8
NRAD = 6
NSPH = 42


_BE = 2000


def _pre_body(x_ref, rbf_ref, wr_ref, wk_ref, bk_ref, m_ref):
    xk = jnp.dot(x_ref[...].astype(jnp.bfloat16), wk_ref[...],
                 preferred_element_type=jnp.float32)
    xk = xk + bk_ref[...]
    rh = jnp.dot(rbf_ref[...], wr_ref[...], preferred_element_type=jnp.float32)
    m_ref[...] = jax.nn.silu(xk) * rh


def _edge_pre(x, rbf, W_rbf, W_kj, b_kj):
    grid = (E // _BE,)
    return pl.pallas_call(
        _pre_body,
        grid=grid,
        in_specs=[
            pl.BlockSpec((_BE, H), lambda i: (i, 0)),
            pl.BlockSpec((_BE, NRAD), lambda i: (i, 0)),
            pl.BlockSpec((NRAD, H), lambda i: (0, 0)),
            pl.BlockSpec((H, H), lambda i: (0, 0)),
            pl.BlockSpec((1, H), lambda i: (0, 0)),
        ],
        out_specs=pl.BlockSpec((_BE, H), lambda i: (i, 0)),
        out_shape=jax.ShapeDtypeStruct((E, H), jnp.float32),
    )(x, rbf, W_rbf, W_kj.astype(jnp.bfloat16), b_kj.reshape(1, H))


_NW = 32
_GPW = T // _NW
_GC = 128
_GFULL = _GPW // _GC
_GTAIL = _GPW - _GFULL * _GC


_NBUF = 6


def _gather_sc(m, idx_kj):
    mesh = plsc.VectorSubcoreMesh(core_axis_name="c", subcore_axis_name="s")

    @functools.partial(
        pl.kernel, mesh=mesh,
        out_type=jax.ShapeDtypeStruct((T, H), jnp.float32),
        compiler_params=pltpu.CompilerParams(needs_layout_passes=False),
        scratch_types=[
            pltpu.VMEM((_NBUF, _GC), jnp.int32),
            pltpu.VMEM((_NBUF, _GC, H), jnp.float32),
            pltpu.SemaphoreType.DMA,
            pltpu.SemaphoreType.DMA,
        ],
    )
    def k(m_hbm, idx_hbm, out_hbm, idx_v, rows_v, gsem, osem):
        c = lax.axis_index("c")
        s = lax.axis_index("s")
        wid = s * 2 + c
        base = wid * _GPW

        def start(j):
            b = lax.rem(j, _NBUF)
            off = base + j * _GC
            pltpu.sync_copy(idx_hbm.at[pl.ds(off, _GC)], idx_v.at[b])
            pltpu.async_copy(m_hbm.at[idx_v.at[b]], rows_v.at[b], gsem)

        def finish(j):
            b = lax.rem(j, _NBUF)
            off = base + j * _GC
            pltpu.make_async_copy(m_hbm.at[idx_v.at[b]], rows_v.at[b],
                                  gsem).wait()
            pltpu.async_copy(rows_v.at[b], out_hbm.at[pl.ds(off, _GC)], osem)

        def drain_out(j):
            b = lax.rem(j, _NBUF)
            off = base + j * _GC
            pltpu.make_async_copy(rows_v.at[b], out_hbm.at[pl.ds(off, _GC)],
                                  osem).wait()

        for j in range(_NBUF - 1):
            start(j)
        finish(0)
        start(_NBUF - 1)

        def chunk(j, _):
            finish(j)
            drain_out(j - 1)
            start(j + _NBUF - 1)
            return 0

        lax.fori_loop(1, _GFULL - (_NBUF - 1), chunk, 0)
        for j in range(_GFULL - (_NBUF - 1), _GFULL):
            finish(j)
            drain_out(j - 1)
        drain_out(_GFULL - 1)
        off = base + _GFULL * _GC
        pltpu.sync_copy(idx_hbm.at[pl.ds(off, _GTAIL)],
                        idx_v.at[0, pl.ds(0, _GTAIL)])
        pltpu.async_copy(m_hbm.at[idx_v.at[0, pl.ds(0, _GTAIL)]],
                         rows_v.at[0, pl.ds(0, _GTAIL)], gsem).wait()
        pltpu.sync_copy(rows_v.at[0, pl.ds(0, _GTAIL)],
                        out_hbm.at[pl.ds(off, _GTAIL)])

    return k(m, idx_kj)


_BT = 3200


def _bil_body(g_ref, sbf_ref, ws_ref, w2_ref, out_ref):
    shb = jnp.dot(sbf_ref[...].astype(jnp.bfloat16), ws_ref[...],
                  preferred_element_type=jnp.float32)
    gb = g_ref[...].astype(jnp.bfloat16)
    tmp = jnp.dot(gb, w2_ref[...], preferred_element_type=jnp.float32)
    y = shb * tmp
    acc = y[:, 0:H]
    for j in range(1, NB):
        acc = acc + y[:, j * H:(j + 1) * H]
    out_ref[...] = acc


def _bilinear(g, sbf, W_sbf_wide, W2r):
    grid = (T // _BT,)
    return pl.pallas_call(
        _bil_body,
        grid=grid,
        in_specs=[
            pl.BlockSpec((_BT, H), lambda i: (i, 0)),
            pl.BlockSpec((_BT, NSPH), lambda i: (i, 0)),
            pl.BlockSpec((NSPH, NB * H), lambda i: (0, 0)),
            pl.BlockSpec((H, NB * H), lambda i: (0, 0)),
        ],
        out_specs=pl.BlockSpec((_BT, H), lambda i: (i, 0)),
        out_shape=jax.ShapeDtypeStruct((T, H), jnp.float32),
    )(g, sbf, W_sbf_wide, W2r)


_RANGE = 7680
_NPASS = 11
_LASTR = (E // 2) - (_NPASS - 1) * _RANGE
_TPT = T // 16
_IDXC = 2000
_ROWS_PT = _RANGE // 16
_LROWS_PT = _LASTR // 16
_LIST = 4608
_FLUSH_HI = _LIST - 208


def _scatter_sc(msg, idx_ji):
    mesh = plsc.VectorSubcoreMesh(core_axis_name="c", subcore_axis_name="s")

    @functools.partial(
        pl.kernel, mesh=mesh,
        out_type=jax.ShapeDtypeStruct((E, H), jnp.float32),
        compiler_params=pltpu.CompilerParams(needs_layout_passes=False),
        scratch_types=[
            pltpu.VMEM((_IDXC,), jnp.int32),
            pltpu.VMEM((_LIST,), jnp.int32),
            pltpu.VMEM((_LIST,), jnp.int32),
            pltpu.VMEM((3, _GC, H), jnp.float32),
            pltpu.VMEM((60, H), jnp.float32),
            pltpu.VMEM_SHARED((_RANGE + 16, H), jnp.float32),
            pltpu.SemaphoreType.DMA,
            pltpu.SemaphoreType.DMA,
        ],
    )
    def k(msg_hbm, idx_hbm, out_hbm, idxb, widl, dstl, rows, zbuf,
          acc, sem, asem):
        c = lax.axis_index("c")
        s = lax.axis_index("s")
        half = c * (E // 2)
        half_end = half + E // 2
        tbase = s * _TPT
        lanes = lax.iota(jnp.int32, 16)

        def zrow(i, _):
            def zcol(kk, __):
                zbuf[i, pl.ds(kk * 16, 16)] = jnp.zeros((16,), jnp.float32)
                return 0
            lax.fori_loop(0, H // 16, zcol, 0)
            return 0
        lax.fori_loop(0, 60, zrow, 0)

        def _if(pred, fn, j):
            lax.cond(pred, lambda _: (fn(j), 0)[1], lambda _: 0, 0)

        def g_start(j):
            b = lax.rem(j, 3)
            pltpu.async_copy(msg_hbm.at[widl.at[pl.ds(j * _GC, _GC)]],
                             rows.at[b], sem)

        def g_wait(j):
            b = lax.rem(j, 3)
            pltpu.make_async_copy(msg_hbm.at[widl.at[pl.ds(j * _GC, _GC)]],
                                  rows.at[b], sem).wait()

        def a_start(j):
            b = lax.rem(j, 3)
            pltpu.async_copy(rows.at[b],
                             acc.at[dstl.at[pl.ds(j * _GC, _GC)]],
                             asem, add=True)

        def a_wait(j):
            b = lax.rem(j, 3)
            pltpu.make_async_copy(rows.at[b],
                                  acc.at[dstl.at[pl.ds(j * _GC, _GC)]],
                                  asem).wait()

        def drain(cnt):
            s_ = lax.axis_index("s")
            lanes_ = lax.iota(jnp.int32, 16)

            def pad(kk, _):
                ppos = cnt + kk * 16 + lanes_
                plsc.store_scatter(widl, [ppos],
                                   s_ * _TPT + kk * 16 + lanes_)
                plsc.store_scatter(dstl, [ppos],
                                   jnp.full((16,), _RANGE + s_, jnp.int32))
                return 0
            lax.fori_loop(0, _GC // 16, pad, 0)
            nch = (cnt + _GC - 1) // _GC
            _if(nch > 0, g_start, 0)

            def proc(k2, _):
                _if(k2 >= 2, a_wait, k2 - 2)
                _if(k2 + 1 < nch, g_start, k2 + 1)
                g_wait(k2)
                a_start(k2)
                return 0
            lax.fori_loop(0, nch, proc, 0)
            _if(nch >= 2, a_wait, nch - 2)
            _if(nch >= 1, a_wait, nch - 1)
            return jnp.int32(0)

        def one_pass(p, _):
            base = half + p * _RANGE
            def zacc(q, __):
                pltpu.sync_copy(zbuf, acc.at[pl.ds(s * _ROWS_PT + q * 60, 60)])
                return 0
            lax.fori_loop(0, _ROWS_PT // 60, zacc, 0)
            plsc.subcore_barrier()

            def chunk(j, cnt):
                pltpu.sync_copy(idx_hbm.at[pl.ds(tbase + j * _IDXC, _IDXC)], idxb)

                def vec(i, cnt):
                    for u in range(5):
                        v = idxb[pl.ds((i * 5 + u) * 16, 16)]
                        rel = v - base
                        msk = (rel >= 0) & (rel < _RANGE) & (v < half_end)
                        w = tbase + j * _IDXC + (i * 5 + u) * 16 + lanes
                        plsc.store_compressed(widl.at[pl.ds(cnt, 16)], w,
                                              mask=msk)
                        plsc.store_compressed(dstl.at[pl.ds(cnt, 16)], rel,
                                              mask=msk)
                        pc = plsc.all_reduce_population_count(msk)
                        cnt = cnt + lax.squeeze(lax.slice(pc, (0,), (1,)), (0,))
                    cnt = lax.cond(cnt > _FLUSH_HI, drain, lambda cc: cc, cnt)
                    return cnt

                return lax.fori_loop(0, _IDXC // 80, vec, cnt)

            cnt = lax.fori_loop(0, _TPT // _IDXC, chunk, jnp.int32(0))
            drain(cnt)
            plsc.subcore_barrier()

            def flush_full(_):
                pltpu.sync_copy(acc.at[pl.ds(s * _ROWS_PT, _ROWS_PT)],
                                out_hbm.at[pl.ds(base + s * _ROWS_PT, _ROWS_PT)])
                return 0

            def flush_last(_):
                pltpu.sync_copy(acc.at[pl.ds(s * _LROWS_PT, _LROWS_PT)],
                                out_hbm.at[pl.ds(base + s * _LROWS_PT, _LROWS_PT)])
                return 0

            lax.cond(p < _NPASS - 1, flush_full, flush_last, 0)
            plsc.subcore_barrier()
            return 0

        lax.fori_loop(0, _NPASS, one_pass, 0)

    return k(msg, idx_ji)


def _final_body(x_ref, agg_ref, wji_ref, bji_ref,
                b0w1_ref, b0b1_ref, b0w2_ref, b0b2_ref,
                wl_ref, bl_ref,
                a0w1_ref, a0b1_ref, a0w2_ref, a0b2_ref,
                a1w1_ref, a1b1_ref, a1w2_ref, a1b2_ref, out_ref):
    act = jax.nn.silu

    def mm(a, w_ref, b_ref):
        return jnp.dot(a.astype(jnp.bfloat16), w_ref[...],
                       preferred_element_type=jnp.float32) + b_ref[...]

    xb = x_ref[...]
    h = act(mm(xb, wji_ref, bji_ref)) + agg_ref[...]
    h = h + act(mm(act(mm(h, b0w1_ref, b0b1_ref)), b0w2_ref, b0b2_ref))
    h = act(mm(h, wl_ref, bl_ref)) + xb
    h = h + act(mm(act(mm(h, a0w1_ref, a0b1_ref)), a0w2_ref, a0b2_ref))
    h = h + act(mm(act(mm(h, a1w1_ref, a1b1_ref)), a1w2_ref, a1b2_ref))
    out_ref[...] = h


def _final(x, agg, W_ji, b_ji, bs0_W1, bs0_b1, bs0_W2, bs0_b2,
           W_lin, b_lin, as0_W1, as0_b1, as0_W2, as0_b2,
           as1_W1, as1_b1, as1_W2, as1_b2):
    grid = (E // _BE,)
    row = pl.BlockSpec((_BE, H), lambda i: (i, 0))
    wsp = pl.BlockSpec((H, H), lambda i: (0, 0))
    bsp = pl.BlockSpec((1, H), lambda i: (0, 0))
    ws = [W_ji, b_ji, bs0_W1, bs0_b1, bs0_W2, bs0_b2, W_lin, b_lin,
          as0_W1, as0_b1, as0_W2, as0_b2, as1_W1, as1_b1, as1_W2, as1_b2]
    specs = []
    wargs = []
    for w in ws:
        if w.ndim == 1:
            specs.append(bsp)
            wargs.append(w.reshape(1, H))
        else:
            specs.append(wsp)
            wargs.append(w.astype(jnp.bfloat16))
    return pl.pallas_call(
        _final_body,
        grid=grid,
        in_specs=[row, row] + specs,
        out_specs=row,
        out_shape=jax.ShapeDtypeStruct((E, H), jnp.float32),
    )(x, agg, *wargs)


def kernel(x, rbf, sbf, idx_kj, idx_ji, angle,
           W_rbf, W_sbf, W_kj, b_kj, W_ji, b_ji, W_bil,
           bs0_W1, bs0_b1, bs0_W2, bs0_b2,
           W_lin, b_lin,
           as0_W1, as0_b1, as0_W2, as0_b2,
           as1_W1, as1_b1, as1_W2, as1_b2):
    idx_kj = idx_kj.astype(jnp.int32)
    idx_ji = idx_ji.astype(jnp.int32)
    W2r = jnp.transpose(W_bil, (2, 1, 0)).reshape(H, NB * H).astype(jnp.bfloat16)
    W_sbf_wide = jnp.kron(W_sbf, jnp.ones((1, H), jnp.float32)).astype(jnp.bfloat16)

    m = _edge_pre(x, rbf, W_rbf, W_kj, b_kj)
    g = _gather_sc(m, idx_kj)
    msg = _bilinear(g, sbf, W_sbf_wide, W2r)
    agg = _scatter_sc(msg, idx_ji)
    return _final(x, agg, W_ji, b_ji, bs0_W1, bs0_b1, bs0_W2, bs0_b2,
                  W_lin, b_lin, as0_W1, as0_b1, as0_W2, as0_b2,
                  as1_W1, as1_b1, as1_W2, as1_b2)


# speedup vs baseline: 1.0276x; 1.0276x over previous
import functools

import jax
import jax.numpy as jnp
from jax import lax
from jax.experimental import pallas as pl
from jax.experimental.pallas import tpu as pltpu
from jax.experimental.pallas import tpu_sc as plsc

E = 160000
T = 320000
H = 128
NB = 8
NRAD = 6
NSPH = 42


_BE = 2000


def _pre_body(x_ref, rbf_ref, wr_ref, wk_ref, bk_ref, m_ref):
    xk = jnp.dot(x_ref[...].astype(jnp.bfloat16), wk_ref[...],
                 preferred_element_type=jnp.float32)
    xk = xk + bk_ref[...]
    rh = jnp.dot(rbf_ref[...], wr_ref[...], preferred_element_type=jnp.float32)
    m_ref[...] = jax.nn.silu(xk) * rh


def _edge_pre(x, rbf, W_rbf, W_kj, b_kj):
    grid = (E // _BE,)
    return pl.pallas_call(
        _pre_body,
        grid=grid,
        in_specs=[
            pl.BlockSpec((_BE, H), lambda i: (i, 0)),
            pl.BlockSpec((_BE, NRAD), lambda i: (i, 0)),
            pl.BlockSpec((NRAD, H), lambda i: (0, 0)),
            pl.BlockSpec((H, H), lambda i: (0, 0)),
            pl.BlockSpec((1, H), lambda i: (0, 0)),
        ],
        out_specs=pl.BlockSpec((_BE, H), lambda i: (i, 0)),
        out_shape=jax.ShapeDtypeStruct((E, H), jnp.float32),
    )(x, rbf, W_rbf, W_kj.astype(jnp.bfloat16), b_kj.reshape(1, H))


_NW = 32
_GPW = T // _NW
_GC = 128
_GFULL = _GPW // _GC
_GTAIL = _GPW - _GFULL * _GC


_NBUF = 6


def _gather_sc(m, idx_kj):
    mesh = plsc.VectorSubcoreMesh(core_axis_name="c", subcore_axis_name="s")

    @functools.partial(
        pl.kernel, mesh=mesh,
        out_type=jax.ShapeDtypeStruct((T, H), jnp.float32),
        compiler_params=pltpu.CompilerParams(needs_layout_passes=False),
        scratch_types=[
            pltpu.VMEM((_NBUF, _GC), jnp.int32),
            pltpu.VMEM((_NBUF, _GC, H), jnp.float32),
            pltpu.SemaphoreType.DMA,
            pltpu.SemaphoreType.DMA,
        ],
    )
    def k(m_hbm, idx_hbm, out_hbm, idx_v, rows_v, gsem, osem):
        c = lax.axis_index("c")
        s = lax.axis_index("s")
        wid = s * 2 + c
        base = wid * _GPW

        def start(j):
            b = lax.rem(j, _NBUF)
            off = base + j * _GC
            pltpu.sync_copy(idx_hbm.at[pl.ds(off, _GC)], idx_v.at[b])
            pltpu.async_copy(m_hbm.at[idx_v.at[b]], rows_v.at[b], gsem)

        def finish(j):
            b = lax.rem(j, _NBUF)
            off = base + j * _GC
            pltpu.make_async_copy(m_hbm.at[idx_v.at[b]], rows_v.at[b],
                                  gsem).wait()
            pltpu.async_copy(rows_v.at[b], out_hbm.at[pl.ds(off, _GC)], osem)

        def drain_out(j):
            b = lax.rem(j, _NBUF)
            off = base + j * _GC
            pltpu.make_async_copy(rows_v.at[b], out_hbm.at[pl.ds(off, _GC)],
                                  osem).wait()

        for j in range(_NBUF - 1):
            start(j)
        finish(0)
        start(_NBUF - 1)

        def chunk(j, _):
            finish(j)
            drain_out(j - 1)
            start(j + _NBUF - 1)
            return 0

        lax.fori_loop(1, _GFULL - (_NBUF - 1), chunk, 0)
        for j in range(_GFULL - (_NBUF - 1), _GFULL):
            finish(j)
            drain_out(j - 1)
        drain_out(_GFULL - 1)
        off = base + _GFULL * _GC
        pltpu.sync_copy(idx_hbm.at[pl.ds(off, _GTAIL)],
                        idx_v.at[0, pl.ds(0, _GTAIL)])
        pltpu.async_copy(m_hbm.at[idx_v.at[0, pl.ds(0, _GTAIL)]],
                         rows_v.at[0, pl.ds(0, _GTAIL)], gsem).wait()
        pltpu.sync_copy(rows_v.at[0, pl.ds(0, _GTAIL)],
                        out_hbm.at[pl.ds(off, _GTAIL)])

    return k(m, idx_kj)


_BT = 3200


def _bil_body(g_ref, sbf_ref, ws_ref, w2_ref, out_ref):
    shb = jnp.dot(sbf_ref[...].astype(jnp.bfloat16), ws_ref[...],
                  preferred_element_type=jnp.float32)
    gb = g_ref[...].astype(jnp.bfloat16)
    tmp = jnp.dot(gb, w2_ref[...], preferred_element_type=jnp.float32)
    y = shb * tmp
    acc = y[:, 0:H]
    for j in range(1, NB):
        acc = acc + y[:, j * H:(j + 1) * H]
    out_ref[...] = acc


def _bilinear(g, sbf, W_sbf_wide, W2r):
    grid = (T // _BT,)
    return pl.pallas_call(
        _bil_body,
        grid=grid,
        in_specs=[
            pl.BlockSpec((_BT, H), lambda i: (i, 0)),
            pl.BlockSpec((_BT, NSPH), lambda i: (i, 0)),
            pl.BlockSpec((NSPH, NB * H), lambda i: (0, 0)),
            pl.BlockSpec((H, NB * H), lambda i: (0, 0)),
        ],
        out_specs=pl.BlockSpec((_BT, H), lambda i: (i, 0)),
        out_shape=jax.ShapeDtypeStruct((T, H), jnp.float32),
    )(g, sbf, W_sbf_wide, W2r)


_RANGE = 9600
_NPASS = 9
_LASTR = (E // 2) - (_NPASS - 1) * _RANGE
_TPT = T // 16
_IDXC = 2000
_ROWS_PT = _RANGE // 16
_LROWS_PT = _LASTR // 16
_LIST = 4480
_FLUSH_HI = _LIST - 208


def _scatter_sc(msg, idx_ji):
    mesh = plsc.VectorSubcoreMesh(core_axis_name="c", subcore_axis_name="s")

    @functools.partial(
        pl.kernel, mesh=mesh,
        out_type=jax.ShapeDtypeStruct((E, H), jnp.float32),
        compiler_params=pltpu.CompilerParams(needs_layout_passes=False),
        scratch_types=[
            pltpu.VMEM((_IDXC,), jnp.int32),
            pltpu.VMEM((_LIST,), jnp.int32),
            pltpu.VMEM((_LIST,), jnp.int32),
            pltpu.VMEM((2, _GC, H), jnp.float32),
            pltpu.VMEM((75, H), jnp.float32),
            pltpu.VMEM_SHARED((_RANGE + 16, H), jnp.float32),
            pltpu.SemaphoreType.DMA,
            pltpu.SemaphoreType.DMA,
        ],
    )
    def k(msg_hbm, idx_hbm, out_hbm, idxb, widl, dstl, rows, zbuf,
          acc, sem, asem):
        c = lax.axis_index("c")
        s = lax.axis_index("s")
        half = c * (E // 2)
        half_end = half + E // 2
        tbase = s * _TPT
        lanes = lax.iota(jnp.int32, 16)

        def zrow(i, _):
            def zcol(kk, __):
                zbuf[i, pl.ds(kk * 16, 16)] = jnp.zeros((16,), jnp.float32)
                return 0
            lax.fori_loop(0, H // 16, zcol, 0)
            return 0
        lax.fori_loop(0, 75, zrow, 0)

        def _if(pred, fn, j):
            lax.cond(pred, lambda _: (fn(j), 0)[1], lambda _: 0, 0)

        def g_start(j):
            b = lax.rem(j, 2)
            pltpu.async_copy(msg_hbm.at[widl.at[pl.ds(j * _GC, _GC)]],
                             rows.at[b], sem)

        def g_wait(j):
            b = lax.rem(j, 2)
            pltpu.make_async_copy(msg_hbm.at[widl.at[pl.ds(j * _GC, _GC)]],
                                  rows.at[b], sem).wait()

        def a_start(j):
            b = lax.rem(j, 2)
            pltpu.async_copy(rows.at[b],
                             acc.at[dstl.at[pl.ds(j * _GC, _GC)]],
                             asem, add=True)

        def a_wait(j):
            b = lax.rem(j, 2)
            pltpu.make_async_copy(rows.at[b],
                                  acc.at[dstl.at[pl.ds(j * _GC, _GC)]],
                                  asem).wait()

        def drain(cnt):
            s_ = lax.axis_index("s")
            lanes_ = lax.iota(jnp.int32, 16)

            def pad(kk, _):
                ppos = cnt + kk * 16 + lanes_
                plsc.store_scatter(widl, [ppos],
                                   s_ * _TPT + kk * 16 + lanes_)
                plsc.store_scatter(dstl, [ppos],
                                   jnp.full((16,), _RANGE + s_, jnp.int32))
                return 0
            lax.fori_loop(0, _GC // 16, pad, 0)
            nch = (cnt + _GC - 1) // _GC
            _if(nch > 0, g_start, 0)

            def proc(k2, _):
                _if(k2 >= 1, a_wait, k2 - 1)
                _if(k2 + 1 < nch, g_start, k2 + 1)
                g_wait(k2)
                a_start(k2)
                return 0
            lax.fori_loop(0, nch, proc, 0)
            _if(nch >= 1, a_wait, nch - 1)
            return jnp.int32(0)

        def one_pass(p, _):
            base = half + p * _RANGE
            def zacc(q, __):
                pltpu.sync_copy(zbuf, acc.at[pl.ds(s * _ROWS_PT + q * 75, 75)])
                return 0
            lax.fori_loop(0, _ROWS_PT // 75, zacc, 0)
            plsc.subcore_barrier()

            def chunk(j, cnt):
                pltpu.sync_copy(idx_hbm.at[pl.ds(tbase + j * _IDXC, _IDXC)], idxb)

                def vec(i, cnt):
                    for u in range(5):
                        v = idxb[pl.ds((i * 5 + u) * 16, 16)]
                        rel = v - base
                        msk = (rel >= 0) & (rel < _RANGE) & (v < half_end)
                        w = tbase + j * _IDXC + (i * 5 + u) * 16 + lanes
                        plsc.store_compressed(widl.at[pl.ds(cnt, 16)], w,
                                              mask=msk)
                        plsc.store_compressed(dstl.at[pl.ds(cnt, 16)], rel,
                                              mask=msk)
                        pc = plsc.all_reduce_population_count(msk)
                        cnt = cnt + lax.squeeze(lax.slice(pc, (0,), (1,)), (0,))
                    cnt = lax.cond(cnt > _FLUSH_HI, drain, lambda cc: cc, cnt)
                    return cnt

                return lax.fori_loop(0, _IDXC // 80, vec, cnt)

            cnt = lax.fori_loop(0, _TPT // _IDXC, chunk, jnp.int32(0))
            drain(cnt)
            plsc.subcore_barrier()

            def flush_full(_):
                pltpu.sync_copy(acc.at[pl.ds(s * _ROWS_PT, _ROWS_PT)],
                                out_hbm.at[pl.ds(base + s * _ROWS_PT, _ROWS_PT)])
                return 0

            def flush_last(_):
                pltpu.sync_copy(acc.at[pl.ds(s * _LROWS_PT, _LROWS_PT)],
                                out_hbm.at[pl.ds(base + s * _LROWS_PT, _LROWS_PT)])
                return 0

            lax.cond(p < _NPASS - 1, flush_full, flush_last, 0)
            plsc.subcore_barrier()
            return 0

        lax.fori_loop(0, _NPASS, one_pass, 0)

    return k(msg, idx_ji)


def _final_body(x_ref, agg_ref, wji_ref, bji_ref,
                b0w1_ref, b0b1_ref, b0w2_ref, b0b2_ref,
                wl_ref, bl_ref,
                a0w1_ref, a0b1_ref, a0w2_ref, a0b2_ref,
                a1w1_ref, a1b1_ref, a1w2_ref, a1b2_ref, out_ref):
    act = jax.nn.silu

    def mm(a, w_ref, b_ref):
        return jnp.dot(a.astype(jnp.bfloat16), w_ref[...],
                       preferred_element_type=jnp.float32) + b_ref[...]

    xb = x_ref[...]
    h = act(mm(xb, wji_ref, bji_ref)) + agg_ref[...]
    h = h + act(mm(act(mm(h, b0w1_ref, b0b1_ref)), b0w2_ref, b0b2_ref))
    h = act(mm(h, wl_ref, bl_ref)) + xb
    h = h + act(mm(act(mm(h, a0w1_ref, a0b1_ref)), a0w2_ref, a0b2_ref))
    h = h + act(mm(act(mm(h, a1w1_ref, a1b1_ref)), a1w2_ref, a1b2_ref))
    out_ref[...] = h


def _final(x, agg, W_ji, b_ji, bs0_W1, bs0_b1, bs0_W2, bs0_b2,
           W_lin, b_lin, as0_W1, as0_b1, as0_W2, as0_b2,
           as1_W1, as1_b1, as1_W2, as1_b2):
    grid = (E // _BE,)
    row = pl.BlockSpec((_BE, H), lambda i: (i, 0))
    wsp = pl.BlockSpec((H, H), lambda i: (0, 0))
    bsp = pl.BlockSpec((1, H), lambda i: (0, 0))
    ws = [W_ji, b_ji, bs0_W1, bs0_b1, bs0_W2, bs0_b2, W_lin, b_lin,
          as0_W1, as0_b1, as0_W2, as0_b2, as1_W1, as1_b1, as1_W2, as1_b2]
    specs = []
    wargs = []
    for w in ws:
        if w.ndim == 1:
            specs.append(bsp)
            wargs.append(w.reshape(1, H))
        else:
            specs.append(wsp)
            wargs.append(w.astype(jnp.bfloat16))
    return pl.pallas_call(
        _final_body,
        grid=grid,
        in_specs=[row, row] + specs,
        out_specs=row,
        out_shape=jax.ShapeDtypeStruct((E, H), jnp.float32),
    )(x, agg, *wargs)


def kernel(x, rbf, sbf, idx_kj, idx_ji, angle,
           W_rbf, W_sbf, W_kj, b_kj, W_ji, b_ji, W_bil,
           bs0_W1, bs0_b1, bs0_W2, bs0_b2,
           W_lin, b_lin,
           as0_W1, as0_b1, as0_W2, as0_b2,
           as1_W1, as1_b1, as1_W2, as1_b2):
    idx_kj = idx_kj.astype(jnp.int32)
    idx_ji = idx_ji.astype(jnp.int32)
    W2r = jnp.transpose(W_bil, (2, 1, 0)).reshape(H, NB * H).astype(jnp.bfloat16)
    W_sbf_wide = jnp.kron(W_sbf, jnp.ones((1, H), jnp.float32)).astype(jnp.bfloat16)

    m = _edge_pre(x, rbf, W_rbf, W_kj, b_kj)
    g = _gather_sc(m, idx_kj)
    msg = _bilinear(g, sbf, W_sbf_wide, W2r)
    agg = _scatter_sc(msg, idx_ji)
    return _final(x, agg, W_ji, b_ji, bs0_W1, bs0_b1, bs0_W2, bs0_b2,
                  W_lin, b_lin, as0_W1, as0_b1, as0_W2, as0_b2,
                  as1_W1, as1_b1, as1_W2, as1_b2)
